# baseline (device time: 23545 ns/iter reference)
import jax
import jax.numpy as jnp
from jax import lax
from jax.experimental import pallas as pl
from jax.experimental.pallas import tpu as pltpu

P = 16
ZP = 4
QP = 4


def kernel(x):
    m, n = x.shape
    mq = m // QP
    mz = mq // ZP

    def body(x_ref, out_ref, xb_ref, a_buf, pr32, pr_ref, b_buf, fr_ref,
             qbuf, snd, rcv):
        my = lax.axis_index("i")
        z = my // QP
        q = my % QP

        barrier = pltpu.get_barrier_semaphore()
        for j in range(1, QP):
            pl.semaphore_signal(
                barrier, inc=1,
                device_id=(z * QP + (q + j) % QP,),
                device_id_type=pl.DeviceIdType.MESH,
            )
        for j in range(1, ZP):
            pl.semaphore_signal(
                barrier, inc=1,
                device_id=(((z + j) % ZP) * QP + q,),
                device_id_type=pl.DeviceIdType.MESH,
            )
        pl.semaphore_wait(barrier, QP - 1 + ZP - 1)

        xb_ref[...] = x_ref[...].astype(jnp.bfloat16)

        sends = []

        for j in range(1, QP):
            pq = (q + j) % QP
            rdma = pltpu.make_async_remote_copy(
                src_ref=xb_ref.at[pl.ds(pq * mq, mq), :],
                dst_ref=a_buf.at[pl.ds(q * mq, mq), :],
                send_sem=snd.at[0, pq],
                recv_sem=rcv.at[0, q],
                device_id=(z * QP + pq,),
                device_id_type=pl.DeviceIdType.MESH,
            )
            rdma.start()
            sends.append(rdma)

        pr32[...] = x_ref[pl.ds(q * mq, mq), :]
        for j in range(1, QP):
            sq = (q - j) % QP
            recv = pltpu.make_async_remote_copy(
                src_ref=xb_ref.at[pl.ds(0, mq), :],
                dst_ref=a_buf.at[pl.ds(sq * mq, mq), :],
                send_sem=snd.at[0, sq],
                recv_sem=rcv.at[0, sq],
                device_id=(my,),
                device_id_type=pl.DeviceIdType.MESH,
            )
            recv.wait_recv()
            pr32[...] = pr32[...] + a_buf[pl.ds(sq * mq, mq), :].astype(
                jnp.float32
            )
        pr_ref[...] = pr32[...].astype(jnp.bfloat16)

        for j in range(1, ZP):
            zt = (z + j) % ZP
            rdma = pltpu.make_async_remote_copy(
                src_ref=pr_ref.at[pl.ds(zt * mz, mz), :],
                dst_ref=b_buf.at[pl.ds(z * mz, mz), :],
                send_sem=snd.at[1, zt],
                recv_sem=rcv.at[1, z],
                device_id=(zt * QP + q,),
                device_id_type=pl.DeviceIdType.MESH,
            )
            rdma.start()
            sends.append(rdma)

        facc = pr32[pl.ds(z * mz, mz), :]
        for j in range(1, ZP):
            sz = (z - j) % ZP
            recv = pltpu.make_async_remote_copy(
                src_ref=pr_ref.at[pl.ds(0, mz), :],
                dst_ref=b_buf.at[pl.ds(sz * mz, mz), :],
                send_sem=snd.at[1, sz],
                recv_sem=rcv.at[1, sz],
                device_id=(my,),
                device_id_type=pl.DeviceIdType.MESH,
            )
            recv.wait_recv()
            facc = facc + b_buf[pl.ds(sz * mz, mz), :].astype(jnp.float32)
        fr_ref[...] = facc.astype(jnp.bfloat16)
        qbuf[pl.ds(z * mz, mz), :] = fr_ref[...]

        for j in range(1, ZP):
            zt = (z + j) % ZP
            rdma = pltpu.make_async_remote_copy(
                src_ref=fr_ref,
                dst_ref=qbuf.at[pl.ds(z * mz, mz), :],
                send_sem=snd.at[2, zt],
                recv_sem=rcv.at[2, z],
                device_id=(zt * QP + q,),
                device_id_type=pl.DeviceIdType.MESH,
            )
            rdma.start()
            sends.append(rdma)

        for j in range(1, ZP):
            sz = (z - j) % ZP
            recv = pltpu.make_async_remote_copy(
                src_ref=fr_ref,
                dst_ref=qbuf.at[pl.ds(sz * mz, mz), :],
                send_sem=snd.at[2, sz],
                recv_sem=rcv.at[2, sz],
                device_id=(my,),
                device_id_type=pl.DeviceIdType.MESH,
            )
            recv.wait_recv()
        out_ref[pl.ds(q * mq, mq), :] = qbuf[...]

        for j in range(1, QP):
            pq = (q + j) % QP
            rdma = pltpu.make_async_remote_copy(
                src_ref=qbuf,
                dst_ref=out_ref.at[pl.ds(q * mq, mq), :],
                send_sem=snd.at[3, pq],
                recv_sem=rcv.at[3, q],
                device_id=(z * QP + pq,),
                device_id_type=pl.DeviceIdType.MESH,
            )
            rdma.start()
            sends.append(rdma)

        for j in range(1, QP):
            sq = (q - j) % QP
            recv = pltpu.make_async_remote_copy(
                src_ref=qbuf,
                dst_ref=out_ref.at[pl.ds(sq * mq, mq), :],
                send_sem=snd.at[3, sq],
                recv_sem=rcv.at[3, sq],
                device_id=(my,),
                device_id_type=pl.DeviceIdType.MESH,
            )
            recv.wait_recv()

        for r in sends:
            r.wait_send()

    return pl.pallas_call(
        body,
        out_shape=jax.ShapeDtypeStruct((m, n), jnp.bfloat16),
        in_specs=[pl.BlockSpec(memory_space=pltpu.VMEM)],
        out_specs=pl.BlockSpec(memory_space=pltpu.VMEM),
        scratch_shapes=[
            pltpu.VMEM((m, n), jnp.bfloat16),
            pltpu.VMEM((m, n), jnp.bfloat16),
            pltpu.VMEM((mq, n), jnp.float32),
            pltpu.VMEM((mq, n), jnp.bfloat16),
            pltpu.VMEM((mq, n), jnp.bfloat16),
            pltpu.VMEM((mz, n), jnp.bfloat16),
            pltpu.VMEM((mq, n), jnp.bfloat16),
            pltpu.SemaphoreType.DMA((4, 4)),
            pltpu.SemaphoreType.DMA((4, 4)),
        ],
        compiler_params=pltpu.CompilerParams(collective_id=0),
    )(x)


# device time: 21336 ns/iter; 1.1035x vs baseline; 1.1035x over previous
import jax
import jax.numpy as jnp
from jax import lax
from jax.experimental import pallas as pl
from jax.experimental.pallas import tpu as pltpu

P = 16


def kernel(x):
    m, n = x.shape
    c = m // P

    def body(x_ref, out_ref, xb_ref, red_ref, rs_buf,
             rs_send, rs_recv, ag_send, ag_recv):
        my = lax.axis_index("i")

        with jax.named_scope("barrier"):
            barrier = pltpu.get_barrier_semaphore()
            for k in range(1, P):
                pl.semaphore_signal(
                    barrier, inc=1,
                    device_id=((my + k) % P,),
                    device_id_type=pl.DeviceIdType.MESH,
                )
            pl.semaphore_wait(barrier, P - 1)

        with jax.named_scope("tobf16"):
            xb_ref[...] = x_ref[...].astype(jnp.bfloat16)

        sends1 = []
        with jax.named_scope("p1_send"):
            for k in range(1, P):
                dst = (my + k) % P
                rdma = pltpu.make_async_remote_copy(
                    src_ref=xb_ref.at[pl.ds(dst * c, c), :],
                    dst_ref=rs_buf.at[pl.ds(my * c, c), :],
                    send_sem=rs_send.at[dst],
                    recv_sem=rs_recv.at[my],
                    device_id=(dst,),
                    device_id_type=pl.DeviceIdType.MESH,
                )
                rdma.start()
                sends1.append(rdma)

        with jax.named_scope("p1_reduce"):
            acc = x_ref[pl.ds(my * c, c), :]
            for k in range(1, P):
                src = (my - k) % P
                recv = pltpu.make_async_remote_copy(
                    src_ref=xb_ref.at[pl.ds(0, c), :],
                    dst_ref=rs_buf.at[pl.ds(src * c, c), :],
                    send_sem=rs_send.at[src],
                    recv_sem=rs_recv.at[src],
                    device_id=(src,),
                    device_id_type=pl.DeviceIdType.MESH,
                )
                recv.wait_recv()
                acc = acc + rs_buf[pl.ds(src * c, c), :].astype(jnp.float32)

            red_ref[...] = acc.astype(jnp.bfloat16)
            out_ref[pl.ds(my * c, c), :] = red_ref[...]

        sends2 = []
        with jax.named_scope("p2_send"):
            for k in range(1, P):
                dst = (my + k) % P
                rdma = pltpu.make_async_remote_copy(
                    src_ref=red_ref,
                    dst_ref=out_ref.at[pl.ds(my * c, c), :],
                    send_sem=ag_send.at[dst],
                    recv_sem=ag_recv.at[my],
                    device_id=(dst,),
                    device_id_type=pl.DeviceIdType.MESH,
                )
                rdma.start()
                sends2.append(rdma)

        with jax.named_scope("p2_wait"):
            for k in range(1, P):
                src = (my - k) % P
                recv = pltpu.make_async_remote_copy(
                    src_ref=red_ref,
                    dst_ref=out_ref.at[pl.ds(src * c, c), :],
                    send_sem=ag_send.at[src],
                    recv_sem=ag_recv.at[src],
                    device_id=(src,),
                    device_id_type=pl.DeviceIdType.MESH,
                )
                recv.wait_recv()

        with jax.named_scope("drain"):
            for r in sends1:
                r.wait_send()
            for r in sends2:
                r.wait_send()

    return pl.pallas_call(
        body,
        out_shape=jax.ShapeDtypeStruct((m, n), jnp.bfloat16),
        in_specs=[pl.BlockSpec(memory_space=pltpu.VMEM)],
        out_specs=pl.BlockSpec(memory_space=pltpu.VMEM),
        scratch_shapes=[
            pltpu.VMEM((m, n), jnp.bfloat16),
            pltpu.VMEM((c, n), jnp.bfloat16),
            pltpu.VMEM((m, n), jnp.bfloat16),
            pltpu.SemaphoreType.DMA((P,)),
            pltpu.SemaphoreType.DMA((P,)),
            pltpu.SemaphoreType.DMA((P,)),
            pltpu.SemaphoreType.DMA((P,)),
        ],
        compiler_params=pltpu.CompilerParams(collective_id=0),
    )(x)
